# baseline (speedup 1.0000x reference)
import functools

import jax
import jax.numpy as jnp
from jax import lax
from jax.experimental import pallas as pl
from jax.experimental.pallas import tpu as pltpu
from jax.experimental.pallas import tpu_sc as plsc

DEG = 16
D = 256
DP = D // 2
NC = 2
NS = 16
NW = NC * NS
SB = 8
IDXC = SB * DEG
LANES = 16
NGRP = DP // LANES
NBUF = 3


def _sc_gather_pool(idx_nbr, idx_self, table, bias, B):
    TCH = B // SB
    CHUNKS = TCH // NW
    REM = TCH % NW
    CB = CHUNKS * SB

    mesh = plsc.VectorSubcoreMesh(core_axis_name="c", subcore_axis_name="s")

    @functools.partial(
        pl.kernel,
        mesh=mesh,
        out_type=jax.ShapeDtypeStruct((B, D), jnp.float32),
        scratch_types=[
            pltpu.VMEM(((CHUNKS + 1) * IDXC,), jnp.int32),
            pltpu.VMEM(((CHUNKS + 1) * SB,), jnp.int32),
            pltpu.VMEM((D,), jnp.float32),
            pltpu.VMEM((NBUF, IDXC, DP), jnp.int32),
            pltpu.VMEM((NBUF, SB, DP), jnp.int32),
            pltpu.VMEM((NBUF, SB, D), jnp.float32),
        ] + [pltpu.SemaphoreType.DMA] * (3 * NBUF),
    )
    def sc_kernel(idxn_hbm, idxs_hbm, tab_hbm, b_hbm, out_hbm,
                  idxn_v, idxs_v, b_v, nb_v, sf_v, o_v, *sems):
        sem_n = sems[:NBUF]
        sem_s = sems[NBUF:2 * NBUF]
        sem_w = sems[2 * NBUF:]
        wid = lax.axis_index("s") * NC + lax.axis_index("c")
        base = wid * CB
        pltpu.sync_copy(idxn_hbm.at[pl.ds(base * DEG, CB * DEG)],
                        idxn_v.at[pl.ds(0, CB * DEG)])
        pltpu.sync_copy(idxs_hbm.at[pl.ds(base, CB)],
                        idxs_v.at[pl.ds(0, CB)])
        pltpu.sync_copy(b_hbm, b_v)

        xrow = (CHUNKS * NW + wid) * SB

        @pl.when(wid < REM)
        def _():
            pltpu.sync_copy(idxn_hbm.at[pl.ds(xrow * DEG, IDXC)],
                            idxn_v.at[pl.ds(CB * DEG, IDXC)])
            pltpu.sync_copy(idxs_hbm.at[pl.ds(xrow, SB)],
                            idxs_v.at[pl.ds(CB, SB)])

        def gathers(g, b):
            return [
                pltpu.make_async_copy(
                    tab_hbm.at[idxn_v.at[pl.ds(g * IDXC, IDXC)]],
                    nb_v.at[b], sem_n[b]),
                pltpu.make_async_copy(
                    tab_hbm.at[idxs_v.at[pl.ds(g * SB, SB)]],
                    sf_v.at[b], sem_s[b]),
            ]

        def out_write_at(row, b):
            return pltpu.make_async_copy(
                o_v.at[b], out_hbm.at[pl.ds(row, SB)], sem_w[b])

        def out_write(g, b):
            return out_write_at(base + g * SB, b)

        def start_gathers(g, b):
            for c in gathers(g, b):
                c.start()

        def accum(b):
            def accum_i(i, c2):
                r0 = i * DEG
                sh16 = jnp.full((LANES,), 16, jnp.int32)
                for m in range(NGRP):
                    col = m * LANES
                    x = sf_v[b, i, pl.ds(col, LANES)]
                    se = lax.bitcast_convert_type(x << sh16, jnp.float32)
                    so = lax.bitcast_convert_type(x, jnp.float32)
                    for j in range(DEG):
                        x = nb_v[b, r0 + j, pl.ds(col, LANES)]
                        se = se + lax.bitcast_convert_type(x << sh16, jnp.float32)
                        so = so + lax.bitcast_convert_type(x, jnp.float32)
                    se = se + b_v[pl.ds(2 * col, LANES)]
                    so = so + b_v[pl.ds(2 * col + LANES, LANES)]
                    o_v[b, i, pl.ds(2 * col, LANES)] = jnp.maximum(se, 0.0)
                    o_v[b, i, pl.ds(2 * col + LANES, LANES)] = jnp.maximum(so, 0.0)
                return c2

            lax.fori_loop(0, SB, accum_i, 0)

        start_gathers(0, 0)
        start_gathers(1, 1)

        def body(k, carry):
            for b in range(NBUF):
                g = k * NBUF + b
                nxt = g + 2
                bn = (b + 2) % NBUF

                @pl.when(nxt < CHUNKS)
                def _(nxt=nxt, bn=bn):
                    start_gathers(nxt, bn)

                @pl.when(g >= NBUF)
                def _(g=g, b=b):
                    out_write(g - NBUF, b).wait()

                for c in gathers(g, b):
                    c.wait()
                accum(b)
                out_write(g, b).start()
            return carry

        lax.fori_loop(0, CHUNKS // NBUF, body, 0)

        @pl.when(wid < REM)
        def _():
            out_write(CHUNKS - NBUF, 0).wait()
            for c in gathers(CHUNKS, 0):
                c.start()
            for c in gathers(CHUNKS, 0):
                c.wait()
            accum(0)
            out_write_at(xrow, 0).start()

        @pl.when(wid >= REM)
        def _():
            out_write(CHUNKS - NBUF, 0).wait()
        for t in range(CHUNKS - NBUF + 1, CHUNKS):
            out_write(t, t % NBUF).wait()

        @pl.when(wid < REM)
        def _():
            out_write_at(xrow, 0).wait()

    return sc_kernel(idx_nbr, idx_self, table, bias)


def _tab_body(feat_ref, w_ref, o_ref):
    acc = jnp.dot(feat_ref[...], w_ref[0],
                  preferred_element_type=jnp.float32)
    lo = pltpu.bitcast(acc[:, :DP].astype(jnp.bfloat16),
                       jnp.uint16).astype(jnp.uint32)
    hi = pltpu.bitcast(acc[:, DP:].astype(jnp.bfloat16),
                       jnp.uint16).astype(jnp.uint32)
    o_ref[...] = pltpu.bitcast((hi << 16) | lo, jnp.int32)


def _tc_tables(features, W_stk, N, BM=1000):
    nb = N // BM
    return pl.pallas_call(
        _tab_body,
        grid=(2, nb),
        in_specs=[
            pl.BlockSpec((BM, D), lambda j, i: (i, 0)),
            pl.BlockSpec((1, D, D), lambda j, i: (j, 0, 0)),
        ],
        out_specs=pl.BlockSpec((BM, DP), lambda j, i: (j * nb + i, 0)),
        out_shape=jax.ShapeDtypeStruct((2 * N, DP), jnp.int32),
    )(features, W_stk)


@jax.jit
def kernel(nodes, neighbors, features, W, b):
    B = nodes.shape[0]
    N = features.shape[0]
    idx_nbr = neighbors.astype(jnp.int32).reshape(-1)
    idx_self = nodes.astype(jnp.int32) + N

    p_ = jnp.arange(DP)
    idx_lo = (p_ // LANES) * 32 + (p_ % LANES)
    col_perm = jnp.concatenate([idx_lo, idx_lo + LANES])

    W_stk = jnp.stack([W[D:] * (1.0 / DEG), W[:D]])[:, :, col_perm]
    table_i32 = _tc_tables(features, W_stk, N)

    return _sc_gather_pool(idx_nbr, idx_self, table_i32, b, B)


# speedup vs baseline: 1.2281x; 1.2281x over previous
import functools

import jax
import jax.numpy as jnp
from jax import lax
from jax.experimental import pallas as pl
from jax.experimental.pallas import tpu as pltpu
from jax.experimental.pallas import tpu_sc as plsc

DEG = 16
D = 256
DP = D // 2
NC = 2
NS = 16
NW = NC * NS
SB = 8
IDXC = SB * DEG
LANES = 16
NGRP = DP // LANES
NBUF = 3


def _sc_gather_pool(idx_nbr, idx_self, table, bias, B):
    TCH = B // SB
    CHUNKS = TCH // NW
    REM = TCH % NW
    CB = CHUNKS * SB

    mesh = plsc.VectorSubcoreMesh(core_axis_name="c", subcore_axis_name="s")

    @functools.partial(
        pl.kernel,
        mesh=mesh,
        out_type=jax.ShapeDtypeStruct((B, D), jnp.float32),
        scratch_types=[
            pltpu.VMEM(((CHUNKS + 1) * IDXC,), jnp.int32),
            pltpu.VMEM(((CHUNKS + 1) * SB,), jnp.int32),
            pltpu.VMEM((D,), jnp.float32),
            pltpu.VMEM((NBUF, IDXC, DP), jnp.int32),
            pltpu.VMEM((NBUF, SB, DP), jnp.int32),
            pltpu.VMEM((NBUF, SB, D), jnp.float32),
        ] + [pltpu.SemaphoreType.DMA] * (3 * NBUF),
    )
    def sc_kernel(idxn_hbm, idxs_hbm, tab_hbm, b_hbm, out_hbm,
                  idxn_v, idxs_v, b_v, nb_v, sf_v, o_v, *sems):
        sem_n = sems[:NBUF]
        sem_s = sems[NBUF:2 * NBUF]
        sem_w = sems[2 * NBUF:]
        wid = lax.axis_index("s") * NC + lax.axis_index("c")
        base = wid * CB
        pltpu.sync_copy(idxn_hbm.at[pl.ds(base * DEG, CB * DEG)],
                        idxn_v.at[pl.ds(0, CB * DEG)])
        pltpu.sync_copy(idxs_hbm.at[pl.ds(base, CB)],
                        idxs_v.at[pl.ds(0, CB)])
        pltpu.sync_copy(b_hbm, b_v)

        xrow = (CHUNKS * NW + wid) * SB

        @pl.when(wid < REM)
        def _():
            pltpu.sync_copy(idxn_hbm.at[pl.ds(xrow * DEG, IDXC)],
                            idxn_v.at[pl.ds(CB * DEG, IDXC)])
            pltpu.sync_copy(idxs_hbm.at[pl.ds(xrow, SB)],
                            idxs_v.at[pl.ds(CB, SB)])

        def gathers(g, b):
            return [
                pltpu.make_async_copy(
                    tab_hbm.at[idxn_v.at[pl.ds(g * IDXC, IDXC)]],
                    nb_v.at[b], sem_n[b]),
                pltpu.make_async_copy(
                    tab_hbm.at[idxs_v.at[pl.ds(g * SB, SB)]],
                    sf_v.at[b], sem_s[b]),
            ]

        def out_write_at(row, b):
            return pltpu.make_async_copy(
                o_v.at[b], out_hbm.at[pl.ds(row, SB)], sem_w[b])

        def out_write(g, b):
            return out_write_at(base + g * SB, b)

        def start_gathers(g, b):
            for c in gathers(g, b):
                c.start()

        def accum(b):
            def accum_i(i, c2):
                r0 = i * DEG
                mask = jnp.full((LANES,), -65536, jnp.int32)
                sh16 = jnp.full((LANES,), 16, jnp.int32)
                for m in range(NGRP):
                    col = m * LANES
                    x = sf_v[b, i, pl.ds(col, LANES)]
                    se = lax.bitcast_convert_type(x << sh16, jnp.float32)
                    so = lax.bitcast_convert_type(x & mask, jnp.float32)
                    for j in range(DEG):
                        x = nb_v[b, r0 + j, pl.ds(col, LANES)]
                        se = se + lax.bitcast_convert_type(x << sh16, jnp.float32)
                        so = so + lax.bitcast_convert_type(x & mask, jnp.float32)
                    se = se + b_v[pl.ds(2 * col, LANES)]
                    so = so + b_v[pl.ds(2 * col + LANES, LANES)]
                    o_v[b, i, pl.ds(2 * col, LANES)] = jnp.maximum(se, 0.0)
                    o_v[b, i, pl.ds(2 * col + LANES, LANES)] = jnp.maximum(so, 0.0)
                return c2

            lax.fori_loop(0, SB, accum_i, 0)

        start_gathers(0, 0)
        start_gathers(1, 1)

        def body(k, carry):
            for b in range(NBUF):
                g = k * NBUF + b
                nxt = g + 2
                bn = (b + 2) % NBUF

                @pl.when(nxt < CHUNKS)
                def _(nxt=nxt, bn=bn):
                    start_gathers(nxt, bn)

                @pl.when(g >= NBUF)
                def _(g=g, b=b):
                    out_write(g - NBUF, b).wait()

                for c in gathers(g, b):
                    c.wait()
                accum(b)
                out_write(g, b).start()
            return carry

        lax.fori_loop(0, CHUNKS // NBUF, body, 0)

        @pl.when(wid < REM)
        def _():
            out_write(CHUNKS - NBUF, 0).wait()
            for c in gathers(CHUNKS, 0):
                c.start()
            for c in gathers(CHUNKS, 0):
                c.wait()
            accum(0)
            out_write_at(xrow, 0).start()

        @pl.when(wid >= REM)
        def _():
            out_write(CHUNKS - NBUF, 0).wait()
        for t in range(CHUNKS - NBUF + 1, CHUNKS):
            out_write(t, t % NBUF).wait()

        @pl.when(wid < REM)
        def _():
            out_write_at(xrow, 0).wait()

    return sc_kernel(idx_nbr, idx_self, table, bias)


def _tab_body(feat_ref, w_ref, o_ref):
    acc = jnp.dot(feat_ref[...], w_ref[0],
                  preferred_element_type=jnp.float32)
    lo = pltpu.bitcast(acc[:, :DP].astype(jnp.bfloat16),
                       jnp.uint16).astype(jnp.uint32)
    hi = pltpu.bitcast(acc[:, DP:].astype(jnp.bfloat16),
                       jnp.uint16).astype(jnp.uint32)
    o_ref[...] = pltpu.bitcast((hi << 16) | lo, jnp.int32)


def _tc_tables(features, W_stk, N, BM=2000):
    nb = N // BM
    return pl.pallas_call(
        _tab_body,
        grid=(nb, 2),
        in_specs=[
            pl.BlockSpec((BM, D), lambda i, j: (i, 0)),
            pl.BlockSpec((1, D, D), lambda i, j: (j, 0, 0)),
        ],
        out_specs=pl.BlockSpec((BM, DP), lambda i, j: (j * nb + i, 0)),
        out_shape=jax.ShapeDtypeStruct((2 * N, DP), jnp.int32),
    )(features, W_stk)


@jax.jit
def kernel(nodes, neighbors, features, W, b):
    B = nodes.shape[0]
    N = features.shape[0]
    idx_nbr = neighbors.astype(jnp.int32).reshape(-1)
    idx_self = nodes.astype(jnp.int32) + N

    p_ = jnp.arange(DP)
    idx_lo = (p_ // LANES) * 32 + (p_ % LANES)
    col_perm = jnp.concatenate([idx_lo, idx_lo + LANES])

    W_stk = jnp.stack([W[D:] * (1.0 / DEG), W[:D]])[:, :, col_perm]
    table_i32 = _tc_tables(features, W_stk, N)

    return _sc_gather_pool(idx_nbr, idx_self, table_i32, b, B)


# speedup vs baseline: 1.2796x; 1.0419x over previous
import functools

import jax
import jax.numpy as jnp
from jax import lax
from jax.experimental import pallas as pl
from jax.experimental.pallas import tpu as pltpu
from jax.experimental.pallas import tpu_sc as plsc

DEG = 16
D = 256
DP = D // 2
NC = 2
NS = 16
NW = NC * NS
SB = 8
IDXC = SB * DEG
LANES = 16
NGRP = DP // LANES
NBUF = 3


def _sc_gather_pool(idx_nbr, idx_self, table, bias, B):
    TCH = B // SB
    CHUNKS = TCH // NW
    REM = TCH % NW
    CB = CHUNKS * SB

    mesh = plsc.VectorSubcoreMesh(core_axis_name="c", subcore_axis_name="s")

    @functools.partial(
        pl.kernel,
        mesh=mesh,
        out_type=jax.ShapeDtypeStruct((B, D), jnp.float32),
        scratch_types=[
            pltpu.VMEM(((CHUNKS + 1) * IDXC,), jnp.int32),
            pltpu.VMEM(((CHUNKS + 1) * SB,), jnp.int32),
            pltpu.VMEM((D,), jnp.float32),
            pltpu.VMEM((NBUF, IDXC, DP), jnp.int32),
            pltpu.VMEM((NBUF, SB, DP), jnp.int32),
            pltpu.VMEM((NBUF, SB, D), jnp.float32),
        ] + [pltpu.SemaphoreType.DMA] * (3 * NBUF),
    )
    def sc_kernel(idxn_hbm, idxs_hbm, tab_hbm, b_hbm, out_hbm,
                  idxn_v, idxs_v, b_v, nb_v, sf_v, o_v, *sems):
        sem_n = sems[:NBUF]
        sem_s = sems[NBUF:2 * NBUF]
        sem_w = sems[2 * NBUF:]
        wid = lax.axis_index("s") * NC + lax.axis_index("c")
        base = wid * CB
        pltpu.sync_copy(idxn_hbm.at[pl.ds(base * DEG, CB * DEG)],
                        idxn_v.at[pl.ds(0, CB * DEG)])
        pltpu.sync_copy(idxs_hbm.at[pl.ds(base, CB)],
                        idxs_v.at[pl.ds(0, CB)])
        pltpu.sync_copy(b_hbm, b_v)

        xrow = (CHUNKS * NW + wid) * SB

        @pl.when(wid < REM)
        def _():
            pltpu.sync_copy(idxn_hbm.at[pl.ds(xrow * DEG, IDXC)],
                            idxn_v.at[pl.ds(CB * DEG, IDXC)])
            pltpu.sync_copy(idxs_hbm.at[pl.ds(xrow, SB)],
                            idxs_v.at[pl.ds(CB, SB)])

        def gathers(g, b):
            return [
                pltpu.make_async_copy(
                    tab_hbm.at[idxn_v.at[pl.ds(g * IDXC, IDXC)]],
                    nb_v.at[b], sem_n[b]),
                pltpu.make_async_copy(
                    tab_hbm.at[idxs_v.at[pl.ds(g * SB, SB)]],
                    sf_v.at[b], sem_s[b]),
            ]

        def out_write_at(row, b):
            return pltpu.make_async_copy(
                o_v.at[b], out_hbm.at[pl.ds(row, SB)], sem_w[b])

        def out_write(g, b):
            return out_write_at(base + g * SB, b)

        def start_gathers(g, b):
            for c in gathers(g, b):
                c.start()

        def accum(b):
            def accum_i(i, c2):
                r0 = i * DEG
                mask = jnp.full((LANES,), -65536, jnp.int32)
                sh16 = jnp.full((LANES,), 16, jnp.int32)
                for m in range(NGRP):
                    col = m * LANES
                    x = sf_v[b, i, pl.ds(col, LANES)]
                    se = lax.bitcast_convert_type(x << sh16, jnp.float32)
                    so = lax.bitcast_convert_type(x & mask, jnp.float32)
                    for j in range(DEG):
                        x = nb_v[b, r0 + j, pl.ds(col, LANES)]
                        se = se + lax.bitcast_convert_type(x << sh16, jnp.float32)
                        so = so + lax.bitcast_convert_type(x & mask, jnp.float32)
                    se = se + b_v[pl.ds(2 * col, LANES)]
                    so = so + b_v[pl.ds(2 * col + LANES, LANES)]
                    o_v[b, i, pl.ds(2 * col, LANES)] = jnp.maximum(se, 0.0)
                    o_v[b, i, pl.ds(2 * col + LANES, LANES)] = jnp.maximum(so, 0.0)
                return c2

            lax.fori_loop(0, SB, accum_i, 0)

        start_gathers(0, 0)
        start_gathers(1, 1)

        def body(k, carry):
            for b in range(NBUF):
                g = k * NBUF + b
                nxt = g + 2
                bn = (b + 2) % NBUF

                @pl.when(nxt < CHUNKS)
                def _(nxt=nxt, bn=bn):
                    start_gathers(nxt, bn)

                @pl.when(g >= NBUF)
                def _(g=g, b=b):
                    out_write(g - NBUF, b).wait()

                for c in gathers(g, b):
                    c.wait()
                accum(b)
                out_write(g, b).start()
            return carry

        lax.fori_loop(0, CHUNKS // NBUF, body, 0)

        @pl.when(wid < REM)
        def _():
            out_write(CHUNKS - NBUF, 0).wait()
            for c in gathers(CHUNKS, 0):
                c.start()
            for c in gathers(CHUNKS, 0):
                c.wait()
            accum(0)
            out_write_at(xrow, 0).start()

        @pl.when(wid >= REM)
        def _():
            out_write(CHUNKS - NBUF, 0).wait()
        for t in range(CHUNKS - NBUF + 1, CHUNKS):
            out_write(t, t % NBUF).wait()

        @pl.when(wid < REM)
        def _():
            out_write_at(xrow, 0).wait()

    return sc_kernel(idx_nbr, idx_self, table, bias)


def _pack(acc):
    lo = pltpu.bitcast(acc[:, :DP].astype(jnp.bfloat16),
                       jnp.uint16).astype(jnp.uint32)
    hi = pltpu.bitcast(acc[:, DP:].astype(jnp.bfloat16),
                       jnp.uint16).astype(jnp.uint32)
    return pltpu.bitcast((hi << 16) | lo, jnp.int32)


def _tab_body(feat_ref, w_ref, o_ref):
    f = feat_ref[...]
    o_ref[0] = _pack(jnp.dot(f, w_ref[0], preferred_element_type=jnp.float32))
    o_ref[1] = _pack(jnp.dot(f, w_ref[1], preferred_element_type=jnp.float32))


def _tc_tables(features, W_stk, N, BM=2000):
    nb = N // BM
    out3 = pl.pallas_call(
        _tab_body,
        grid=(nb,),
        in_specs=[
            pl.BlockSpec((BM, D), lambda i: (i, 0)),
            pl.BlockSpec((2, D, D), lambda i: (0, 0, 0)),
        ],
        out_specs=pl.BlockSpec((2, BM, DP), lambda i: (0, i, 0)),
        out_shape=jax.ShapeDtypeStruct((2, N, DP), jnp.int32),
    )(features, W_stk)
    return out3.reshape(2 * N, DP)


@jax.jit
def kernel(nodes, neighbors, features, W, b):
    B = nodes.shape[0]
    N = features.shape[0]
    idx_nbr = neighbors.astype(jnp.int32).reshape(-1)
    idx_self = nodes.astype(jnp.int32) + N

    p_ = jnp.arange(DP)
    idx_lo = (p_ // LANES) * 32 + (p_ % LANES)
    col_perm = jnp.concatenate([idx_lo, idx_lo + LANES])

    W_stk = jnp.stack([W[D:] * (1.0 / DEG), W[:D]])[:, :, col_perm]
    table_i32 = _tc_tables(features, W_stk, N)

    return _sc_gather_pool(idx_nbr, idx_self, table_i32, b, B)


# speedup vs baseline: 1.3047x; 1.0196x over previous
import functools

import jax
import jax.numpy as jnp
from jax import lax
from jax.experimental import pallas as pl
from jax.experimental.pallas import tpu as pltpu
from jax.experimental.pallas import tpu_sc as plsc

DEG = 16
D = 256
DP = D // 2
NC = 2
NS = 16
NW = NC * NS
SB = 8
IDXC = SB * DEG
LANES = 16
NGRP = DP // LANES
NBUF = 3


def _sc_gather_pool(idx_nbr, idx_self, table, bias, B):
    TCH = B // SB
    CHUNKS = TCH // NW
    REM = TCH % NW
    CB = CHUNKS * SB

    mesh = plsc.VectorSubcoreMesh(core_axis_name="c", subcore_axis_name="s")

    @functools.partial(
        pl.kernel,
        mesh=mesh,
        out_type=jax.ShapeDtypeStruct((B, D), jnp.float32),
        scratch_types=[
            pltpu.VMEM(((CHUNKS + 1) * IDXC,), jnp.int32),
            pltpu.VMEM(((CHUNKS + 1) * SB,), jnp.int32),
            pltpu.VMEM((D,), jnp.float32),
            pltpu.VMEM((NBUF, IDXC, DP), jnp.int32),
            pltpu.VMEM((NBUF, SB, DP), jnp.int32),
            pltpu.VMEM((NBUF, SB, D), jnp.float32),
        ] + [pltpu.SemaphoreType.DMA] * (3 * NBUF),
    )
    def sc_kernel(idxn_hbm, idxs_hbm, tab_hbm, b_hbm, out_hbm,
                  idxn_v, idxs_v, b_v, nb_v, sf_v, o_v, *sems):
        sem_n = sems[:NBUF]
        sem_s = sems[NBUF:2 * NBUF]
        sem_w = sems[2 * NBUF:]
        wid = lax.axis_index("s") * NC + lax.axis_index("c")
        base = wid * CB
        pltpu.sync_copy(idxn_hbm.at[pl.ds(base * DEG, CB * DEG)],
                        idxn_v.at[pl.ds(0, CB * DEG)])
        pltpu.sync_copy(idxs_hbm.at[pl.ds(base, CB)],
                        idxs_v.at[pl.ds(0, CB)])
        pltpu.sync_copy(b_hbm, b_v)

        xrow = (CHUNKS * NW + wid) * SB

        @pl.when(wid < REM)
        def _():
            pltpu.sync_copy(idxn_hbm.at[pl.ds(xrow * DEG, IDXC)],
                            idxn_v.at[pl.ds(CB * DEG, IDXC)])
            pltpu.sync_copy(idxs_hbm.at[pl.ds(xrow, SB)],
                            idxs_v.at[pl.ds(CB, SB)])

        def gathers(g, b):
            return [
                pltpu.make_async_copy(
                    tab_hbm.at[idxn_v.at[pl.ds(g * IDXC, IDXC)]],
                    nb_v.at[b], sem_n[b]),
                pltpu.make_async_copy(
                    tab_hbm.at[idxs_v.at[pl.ds(g * SB, SB)]],
                    sf_v.at[b], sem_s[b]),
            ]

        def out_write_at(row, b):
            return pltpu.make_async_copy(
                o_v.at[b], out_hbm.at[pl.ds(row, SB)], sem_w[b])

        def out_write(g, b):
            return out_write_at(base + g * SB, b)

        def start_gathers(g, b):
            for c in gathers(g, b):
                c.start()

        def accum(b):
            def accum_i(i, c2):
                r0 = i * DEG
                mask = jnp.full((LANES,), -65536, jnp.int32)
                sh16 = jnp.full((LANES,), 16, jnp.int32)
                for m in range(NGRP):
                    col = m * LANES
                    x = sf_v[b, i, pl.ds(col, LANES)]
                    se = lax.bitcast_convert_type(x << sh16, jnp.float32)
                    so = lax.bitcast_convert_type(x & mask, jnp.float32)
                    for j in range(DEG):
                        x = nb_v[b, r0 + j, pl.ds(col, LANES)]
                        se = se + lax.bitcast_convert_type(x << sh16, jnp.float32)
                        so = so + lax.bitcast_convert_type(x & mask, jnp.float32)
                    se = se + b_v[pl.ds(2 * col, LANES)]
                    so = so + b_v[pl.ds(2 * col + LANES, LANES)]
                    o_v[b, i, pl.ds(2 * col, LANES)] = jnp.maximum(se, 0.0)
                    o_v[b, i, pl.ds(2 * col + LANES, LANES)] = jnp.maximum(so, 0.0)
                return c2

            lax.fori_loop(0, SB, accum_i, 0)

        start_gathers(0, 0)
        start_gathers(1, 1)

        def body(k, carry):
            for b in range(NBUF):
                g = k * NBUF + b
                nxt = g + 2
                bn = (b + 2) % NBUF

                @pl.when(nxt < CHUNKS)
                def _(nxt=nxt, bn=bn):
                    start_gathers(nxt, bn)

                @pl.when(g >= NBUF)
                def _(g=g, b=b):
                    out_write(g - NBUF, b).wait()

                for c in gathers(g, b):
                    c.wait()
                accum(b)
                out_write(g, b).start()
            return carry

        lax.fori_loop(0, CHUNKS // NBUF, body, 0)

        @pl.when(wid < REM)
        def _():
            out_write(CHUNKS - NBUF, 0).wait()
            for c in gathers(CHUNKS, 0):
                c.start()
            for c in gathers(CHUNKS, 0):
                c.wait()
            accum(0)
            out_write_at(xrow, 0).start()

        @pl.when(wid >= REM)
        def _():
            out_write(CHUNKS - NBUF, 0).wait()
        for t in range(CHUNKS - NBUF + 1, CHUNKS):
            out_write(t, t % NBUF).wait()

        @pl.when(wid < REM)
        def _():
            out_write_at(xrow, 0).wait()

    return sc_kernel(idx_nbr, idx_self, table, bias)


def _pack(acc):
    lo = pltpu.bitcast(acc[:, :DP].astype(jnp.bfloat16),
                       jnp.uint16).astype(jnp.uint32)
    hi = pltpu.bitcast(acc[:, DP:].astype(jnp.bfloat16),
                       jnp.uint16).astype(jnp.uint32)
    return pltpu.bitcast((hi << 16) | lo, jnp.int32)


def _tab_body(feat_ref, w_ref, o_ref):
    f = feat_ref[...]
    o_ref[0] = _pack(jnp.dot(f, w_ref[0], preferred_element_type=jnp.float32))
    o_ref[1] = _pack(jnp.dot(f, w_ref[1], preferred_element_type=jnp.float32))


def _tc_tables(features, W_stk, N, BM=5000):
    nb = N // BM
    out3 = pl.pallas_call(
        _tab_body,
        grid=(nb,),
        in_specs=[
            pl.BlockSpec((BM, D), lambda i: (i, 0)),
            pl.BlockSpec((2, D, D), lambda i: (0, 0, 0)),
        ],
        out_specs=pl.BlockSpec((2, BM, DP), lambda i: (0, i, 0)),
        out_shape=jax.ShapeDtypeStruct((2, N, DP), jnp.int32),
    )(features, W_stk)
    return out3.reshape(2 * N, DP)


@jax.jit
def kernel(nodes, neighbors, features, W, b):
    B = nodes.shape[0]
    N = features.shape[0]
    idx_nbr = neighbors.astype(jnp.int32).reshape(-1)
    idx_self = nodes.astype(jnp.int32) + N

    p_ = jnp.arange(DP)
    idx_lo = (p_ // LANES) * 32 + (p_ % LANES)
    col_perm = jnp.concatenate([idx_lo, idx_lo + LANES])

    W_stk = jnp.stack([W[D:] * (1.0 / DEG), W[:D]])[:, :, col_perm]
    table_i32 = _tc_tables(features, W_stk, N)

    return _sc_gather_pool(idx_nbr, idx_self, table_i32, b, B)
